# baseline (speedup 1.0000x reference)
import functools

import jax
import jax.numpy as jnp
from jax import lax
from jax.experimental import pallas as pl
from jax.experimental.pallas import tpu as pltpu
from jax.experimental.pallas import tpu_sc as plsc

N = 100000
E = 3200000
D_IN = 16
H = 64

NC = 2
NS = 16
NW = NC * NS

N_ACC = 102400
RPT = N_ACC // NS
ZCH = 400
NPK = N_ACC // 8
NDG = N_ACC // 128

CH = 512
CHR = CH // 128
E_TOT = E + N
EW = 103424
E_PAD = EW * NW
WROWS = EW // 128
G = EW // CH
RROWS = E // 128
DTCH = RROWS // CHR

BP = 1600
GRID = NPK // BP

_mesh = plsc.VectorSubcoreMesh(core_axis_name="c", subcore_axis_name="s")


@functools.partial(
    pl.kernel,
    out_type=jax.ShapeDtypeStruct((NC, N_ACC), jnp.float32),
    mesh=_mesh,
    scratch_types=[
        pltpu.VMEM((CHR, 128), jnp.int32),
        pltpu.VMEM((128,), jnp.float32),
        pltpu.VMEM((3200,), jnp.float32),
        pltpu.VMEM_SHARED((N_ACC,), jnp.float32),
        pltpu.SemaphoreType.DMA,
    ],
    compiler_params=pltpu.CompilerParams(use_tc_tiling_on_sc=False),
)
def _deg_kernel(dst_hbm, ones_hbm, zeros_hbm, out_hbm,
                dst_v, ones_v, buf_v, acc_sh, sem):
    c = lax.axis_index("c")
    s = lax.axis_index("s")
    w = c * NS + s

    pltpu.sync_copy(ones_hbm, ones_v)
    pltpu.sync_copy(zeros_hbm, buf_v)

    def zero(k, _):
        pltpu.sync_copy(buf_v, acc_sh.at[pl.ds(s * RPT + k * 3200, 3200)])
        return 0
    lax.fori_loop(0, RPT // 3200, zero, 0)
    plsc.subcore_barrier()

    def body(g, _):
        rb = w * WROWS + g * CHR
        pltpu.sync_copy(dst_hbm.at[pl.ds(rb, CHR)], dst_v)
        cps = [
            pltpu.async_copy(ones_v, acc_sh.at[dst_v.at[j]], sem, add=True)
            for j in range(CHR)
        ]
        for cp in cps:
            cp.wait()
        return 0
    lax.fori_loop(0, G, body, 0)
    plsc.subcore_barrier()

    def wout(k, _):
        base = s * RPT + k * 3200
        pltpu.sync_copy(acc_sh.at[pl.ds(base, 3200)], buf_v)
        pltpu.sync_copy(buf_v, out_hbm.at[c].at[pl.ds(base, 3200)])
        return 0
    lax.fori_loop(0, RPT // 3200, wout, 0)


def _make_agg(width):
    @functools.partial(
        pl.kernel,
        out_type=jax.ShapeDtypeStruct((NC, N_ACC, width), jnp.float32),
        mesh=_mesh,
        scratch_types=[
            pltpu.VMEM((2, CHR, 128), jnp.int32),
            pltpu.VMEM((2, CHR, 128), jnp.int32),
            pltpu.VMEM((2, CH, width), jnp.float32),
            pltpu.VMEM_SHARED((N_ACC, width), jnp.float32),
            pltpu.SemaphoreType.DMA((2,)),
        ],
        compiler_params=pltpu.CompilerParams(use_tc_tiling_on_sc=False),
    )
    def _agg(src_hbm, dst_hbm, val_hbm, zeros_hbm, out_hbm,
             src_v, dst_v, rows_v, acc_sh, sems):
        c = lax.axis_index("c")
        s = lax.axis_index("s")
        w = c * NS + s
        buf_v = rows_v.at[0].at[pl.ds(0, ZCH)]

        pltpu.sync_copy(zeros_hbm, buf_v)

        def zero(k, _):
            pltpu.sync_copy(buf_v, acc_sh.at[pl.ds(s * RPT + k * ZCH, ZCH)])
            return 0
        lax.fori_loop(0, RPT // ZCH, zero, 0)
        plsc.subcore_barrier()

        def load_and_fire(g, bank):
            rb = w * WROWS + g * CHR
            pltpu.sync_copy(src_hbm.at[pl.ds(rb, CHR)], src_v.at[bank])
            pltpu.sync_copy(dst_hbm.at[pl.ds(rb, CHR)], dst_v.at[bank])
            for j in range(CHR):
                pltpu.async_copy(val_hbm.at[src_v.at[bank].at[j]],
                                 rows_v.at[bank].at[pl.ds(j * 128, 128)],
                                 sems.at[bank])

        load_and_fire(0, 0)

        def body(g, _):
            b = lax.rem(g, 2)

            @pl.when(g + 1 < G)
            def _prefetch():
                load_and_fire(g + 1, 1 - b)

            pltpu.make_async_copy(val_hbm.at[pl.ds(0, CH)],
                                  rows_v.at[b], sems.at[b]).wait()
            for j in range(CHR):
                pltpu.sync_copy(rows_v.at[b].at[pl.ds(j * 128, 128)],
                                acc_sh.at[dst_v.at[b].at[j]], add=True)
            return 0
        lax.fori_loop(0, G, body, 0)
        plsc.subcore_barrier()

        def wout(k, _):
            base = s * RPT + k * ZCH
            pltpu.sync_copy(acc_sh.at[pl.ds(base, ZCH)], buf_v)
            pltpu.sync_copy(buf_v, out_hbm.at[c].at[pl.ds(base, ZCH)])
            return 0
        lax.fori_loop(0, RPT // ZCH, wout, 0)

    return _agg


_agg_x = _make_agg(D_IN)
_agg_p = _make_agg(8)


def _dinv_body(deg_ref, dinv_ref):
    dd = deg_ref[...]
    deg = dd[0] + dd[1]
    node = (lax.broadcasted_iota(jnp.int32, (NDG, 128), 0) * 128
            + lax.broadcasted_iota(jnp.int32, (NDG, 128), 1))
    dinv_ref[...] = jnp.where(node < N, lax.rsqrt(deg), 0.0)


def _dinv_call(degp):
    return pl.pallas_call(
        _dinv_body,
        out_shape=jax.ShapeDtypeStruct((NDG, 128), jnp.float32),
    )(degp)


def _prep_body(dinv8_ref, x_ref, k_ref, dinv16_ref, xp_ref):
    d16 = lax.dot_general(dinv8_ref[...], k_ref[...], (((1,), (0,)), ((), ())),
                          preferred_element_type=jnp.float32)
    dinv16_ref[...] = d16
    xp_ref[...] = d16 * x_ref[...]


def _prep_call(dinv8, x_packed, kmat):
    return pl.pallas_call(
        _prep_body,
        grid=(GRID,),
        in_specs=[
            pl.BlockSpec((BP, 8), lambda i: (i, 0)),
            pl.BlockSpec((BP, 128), lambda i: (i, 0)),
            pl.BlockSpec((8, 128), lambda i: (0, 0)),
        ],
        out_specs=[
            pl.BlockSpec((BP, 128), lambda i: (i, 0)),
            pl.BlockSpec((BP, 128), lambda i: (i, 0)),
        ],
        out_shape=[
            jax.ShapeDtypeStruct((NPK, 128), jnp.float32),
            jax.ShapeDtypeStruct((NPK, 128), jnp.float32),
        ],
    )(dinv8, x_packed, kmat)


def _mid_body(agg_ref, dinv_ref, m_ref, w2_ref, b1_ref, out_ref):
    a = agg_ref[...]
    s = dinv_ref[...] * (a[0] + a[1])
    hs = []
    for q in range(4):
        z = lax.dot_general(s, m_ref[q], (((1,), (0,)), ((), ())),
                            preferred_element_type=jnp.float32) + b1_ref[...]
        hs.append(jnp.maximum(z, 0.0))
    hcat = jnp.concatenate(hs, axis=1)
    p = lax.dot_general(hcat, w2_ref[...], (((1,), (0,)), ((), ())),
                        preferred_element_type=jnp.float32)
    out_ref[...] = dinv_ref[...] * p


def _mid_call(agg1, dinv16, mstack, w2pp, b1row):
    return pl.pallas_call(
        _mid_body,
        grid=(GRID,),
        in_specs=[
            pl.BlockSpec((NC, BP, 128), lambda i: (0, i, 0)),
            pl.BlockSpec((BP, 128), lambda i: (i, 0)),
            pl.BlockSpec((4, 128, 128), lambda i: (0, 0, 0)),
            pl.BlockSpec((512, 128), lambda i: (0, 0)),
            pl.BlockSpec((1, 128), lambda i: (0, 0)),
        ],
        out_specs=pl.BlockSpec((BP, 128), lambda i: (i, 0)),
        out_shape=jax.ShapeDtypeStruct((NPK, 128), jnp.float32),
    )(agg1, dinv16, mstack, w2pp, b1row)


def _final_body(agg_ref, dinv_ref, k8_ref, sel_ref, b2_ref, out_ref):
    a = agg_ref[...]
    d8 = lax.dot_general(dinv_ref[...], k8_ref[...], (((1,), (0,)), ((), ())),
                         preferred_element_type=jnp.float32)
    y = lax.dot_general(d8 * (a[0] + a[1]), sel_ref[...],
                        (((1,), (0,)), ((), ())),
                        preferred_element_type=jnp.float32)
    out_ref[...] = y + b2_ref[...]


def _final_call(agg2, dinv16c, k8, sel, b2row32):
    npf = N_ACC // 16
    bpf = npf // GRID
    return pl.pallas_call(
        _final_body,
        grid=(GRID,),
        in_specs=[
            pl.BlockSpec((NC, bpf, 128), lambda i: (0, i, 0)),
            pl.BlockSpec((bpf, 16), lambda i: (i, 0)),
            pl.BlockSpec((16, 128), lambda i: (0, 0)),
            pl.BlockSpec((128, 32), lambda i: (0, 0)),
            pl.BlockSpec((1, 32), lambda i: (0, 0)),
        ],
        out_specs=pl.BlockSpec((bpf, 32), lambda i: (i, 0)),
        out_shape=jax.ShapeDtypeStruct((npf, 32), jnp.float32),
    )(agg2, dinv16c, k8, sel, b2row32)


def kernel(x, edge_index, W1, b1, W2, b2):
    loop = jnp.arange(N, dtype=jnp.int32)
    srcA = jnp.full((E_PAD,), N, jnp.int32).at[:E].set(edge_index[0]) \
        .at[E:E + N].set(loop)
    dstA = jnp.full((E_PAD,), N, jnp.int32).at[:E].set(edge_index[1]) \
        .at[E:E + N].set(loop)
    src1_2d = srcA.reshape(-1, 128)
    src2_2d = (srcA * 2).reshape(-1, 128)
    dst2d = dstA.reshape(-1, 128)

    x_packed = jnp.pad(x.reshape(N // 8, 128), ((0, NPK - N // 8), (0, 0)))

    kmat = jnp.kron(jnp.eye(8, dtype=jnp.float32),
                    jnp.ones((1, 16), jnp.float32))
    mstack = jnp.zeros((4, 128, 128), jnp.float32)
    for q in range(4):
        mstack = mstack.at[q, 32 * q:32 * q + 16, 0:64].set(W1)
        mstack = mstack.at[q, 32 * q + 16:32 * q + 32, 64:128].set(W1)
    w2pp = jnp.zeros((512, 128), jnp.float32)
    for q in range(4):
        for c in range(2):
            a_ = 2 * q + c
            w2pp = w2pp.at[128 * q + 64 * c:128 * q + 64 * c + 64,
                           16 * a_:16 * a_ + 2].set(W2)
    k8 = jnp.kron(jnp.eye(16, dtype=jnp.float32),
                  jnp.ones((1, 8), jnp.float32))
    sel = jnp.zeros((128, 32), jnp.float32)
    for a_ in range(16):
        for b_ in range(2):
            sel = sel.at[8 * a_ + b_, 2 * a_ + b_].set(1.0)
    b1row = jnp.tile(b1, 2).reshape(1, 128)
    b2row32 = jnp.tile(b2, 16).reshape(1, 32)

    ones128 = jnp.ones((128,), jnp.float32)
    zeros3200 = jnp.zeros((3200,), jnp.float32)
    zrows16 = jnp.zeros((ZCH, D_IN), jnp.float32)
    zrows8 = jnp.zeros((ZCH, 8), jnp.float32)

    degp = _deg_kernel(dst2d, ones128, zeros3200)
    dinvf = _dinv_call(degp.reshape(NC, NDG, 128))
    dinv16, xp = _prep_call(dinvf.reshape(NPK, 8), x_packed, kmat)

    agg1 = _agg_x(src1_2d, dst2d, xp.reshape(N_ACC, D_IN), zrows16)
    P2 = _mid_call(agg1.reshape(NC, NPK, 128), dinv16, mstack, w2pp, b1row)

    agg2 = _agg_p(src2_2d, dst2d, P2.reshape(2 * N_ACC, 8), zrows8)
    out = _final_call(agg2.reshape(NC, N_ACC // 16, 128),
                      dinvf.reshape(N_ACC // 16, 16), k8, sel, b2row32)
    return out[:N // 16].reshape(N, 2)


# speedup vs baseline: 1.0132x; 1.0132x over previous
import functools

import jax
import jax.numpy as jnp
from jax import lax
from jax.experimental import pallas as pl
from jax.experimental.pallas import tpu as pltpu
from jax.experimental.pallas import tpu_sc as plsc

N = 100000
E = 3200000
D_IN = 16
H = 64

NC = 2
NS = 16
NW = NC * NS

N_ACC = 102400
RPT = N_ACC // NS
ZCH = 400
NPK = N_ACC // 8
NDG = N_ACC // 128

CH = 512
CHR = CH // 128
E_TOT = E + N
EW = 103424
E_PAD = EW * NW
WROWS = EW // 128
G = EW // CH
RROWS = E // 128
DTCH = RROWS // CHR

BP = 1600
GRID = NPK // BP

_mesh = plsc.VectorSubcoreMesh(core_axis_name="c", subcore_axis_name="s")


@functools.partial(
    pl.kernel,
    out_type=jax.ShapeDtypeStruct((NC, N_ACC), jnp.float32),
    mesh=_mesh,
    scratch_types=[
        pltpu.VMEM((CHR, 128), jnp.int32),
        pltpu.VMEM((128,), jnp.float32),
        pltpu.VMEM((3200,), jnp.float32),
        pltpu.VMEM_SHARED((N_ACC,), jnp.float32),
        pltpu.SemaphoreType.DMA,
    ],
    compiler_params=pltpu.CompilerParams(use_tc_tiling_on_sc=False),
)
def _deg_kernel(dst_hbm, ones_hbm, zeros_hbm, out_hbm,
                dst_v, ones_v, buf_v, acc_sh, sem):
    c = lax.axis_index("c")
    s = lax.axis_index("s")
    w = c * NS + s

    pltpu.sync_copy(ones_hbm, ones_v)
    pltpu.sync_copy(zeros_hbm, buf_v)

    def zero(k, _):
        pltpu.sync_copy(buf_v, acc_sh.at[pl.ds(s * RPT + k * 3200, 3200)])
        return 0
    lax.fori_loop(0, RPT // 3200, zero, 0)
    plsc.subcore_barrier()

    gbase = (DTCH // NW) * w + jnp.minimum(w, DTCH % NW)
    gcnt = DTCH // NW + jnp.where(w < DTCH % NW, 1, 0)

    def body(g, _):
        rb = (gbase + g) * CHR
        pltpu.sync_copy(dst_hbm.at[pl.ds(rb, CHR)], dst_v)
        cps = [
            pltpu.async_copy(ones_v, acc_sh.at[dst_v.at[j]], sem, add=True)
            for j in range(CHR)
        ]
        for cp in cps:
            cp.wait()
        return 0
    lax.fori_loop(0, gcnt, body, 0)
    plsc.subcore_barrier()

    def wout(k, _):
        base = s * RPT + k * 3200
        pltpu.sync_copy(acc_sh.at[pl.ds(base, 3200)], buf_v)
        pltpu.sync_copy(buf_v, out_hbm.at[c].at[pl.ds(base, 3200)])
        return 0
    lax.fori_loop(0, RPT // 3200, wout, 0)


def _make_agg(width):
    @functools.partial(
        pl.kernel,
        out_type=jax.ShapeDtypeStruct((NC, N_ACC, width), jnp.float32),
        mesh=_mesh,
        scratch_types=[
            pltpu.VMEM((2, CHR, 128), jnp.int32),
            pltpu.VMEM((2, CHR, 128), jnp.int32),
            pltpu.VMEM((2, CH, width), jnp.float32),
            pltpu.VMEM_SHARED((N_ACC, width), jnp.float32),
            pltpu.SemaphoreType.DMA((2,)),
        ],
        compiler_params=pltpu.CompilerParams(use_tc_tiling_on_sc=False),
    )
    def _agg(src_hbm, dst_hbm, val_hbm, zeros_hbm, out_hbm,
             src_v, dst_v, rows_v, acc_sh, sems):
        c = lax.axis_index("c")
        s = lax.axis_index("s")
        w = c * NS + s
        buf_v = rows_v.at[0].at[pl.ds(0, ZCH)]

        pltpu.sync_copy(zeros_hbm, buf_v)

        def zero(k, _):
            pltpu.sync_copy(buf_v, acc_sh.at[pl.ds(s * RPT + k * ZCH, ZCH)])
            return 0
        lax.fori_loop(0, RPT // ZCH, zero, 0)
        plsc.subcore_barrier()

        def load_and_fire(g, bank):
            rb = w * WROWS + g * CHR
            pltpu.sync_copy(src_hbm.at[pl.ds(rb, CHR)], src_v.at[bank])
            pltpu.sync_copy(dst_hbm.at[pl.ds(rb, CHR)], dst_v.at[bank])
            for j in range(CHR):
                pltpu.async_copy(val_hbm.at[src_v.at[bank].at[j]],
                                 rows_v.at[bank].at[pl.ds(j * 128, 128)],
                                 sems.at[bank])

        load_and_fire(0, 0)

        def body(g, _):
            b = lax.rem(g, 2)

            @pl.when(g + 1 < G)
            def _prefetch():
                load_and_fire(g + 1, 1 - b)

            pltpu.make_async_copy(val_hbm.at[pl.ds(0, CH)],
                                  rows_v.at[b], sems.at[b]).wait()
            for j in range(CHR):
                pltpu.sync_copy(rows_v.at[b].at[pl.ds(j * 128, 128)],
                                acc_sh.at[dst_v.at[b].at[j]], add=True)
            return 0
        lax.fori_loop(0, G, body, 0)
        plsc.subcore_barrier()

        def wout(k, _):
            base = s * RPT + k * ZCH
            pltpu.sync_copy(acc_sh.at[pl.ds(base, ZCH)], buf_v)
            pltpu.sync_copy(buf_v, out_hbm.at[c].at[pl.ds(base, ZCH)])
            return 0
        lax.fori_loop(0, RPT // ZCH, wout, 0)

    return _agg


_agg_x = _make_agg(D_IN)
_agg_p = _make_agg(8)


def _dinv_body(deg_ref, dinv_ref):
    dd = deg_ref[...]
    deg = dd[0] + dd[1] + 1.0
    node = (lax.broadcasted_iota(jnp.int32, (NDG, 128), 0) * 128
            + lax.broadcasted_iota(jnp.int32, (NDG, 128), 1))
    dinv_ref[...] = jnp.where(node < N, lax.rsqrt(deg), 0.0)


def _dinv_call(degp):
    return pl.pallas_call(
        _dinv_body,
        out_shape=jax.ShapeDtypeStruct((NDG, 128), jnp.float32),
    )(degp)


def _prep_body(dinv8_ref, x_ref, k_ref, dinv16_ref, xp_ref):
    d16 = lax.dot_general(dinv8_ref[...], k_ref[...], (((1,), (0,)), ((), ())),
                          preferred_element_type=jnp.float32)
    dinv16_ref[...] = d16
    xp_ref[...] = d16 * x_ref[...]


def _prep_call(dinv8, x_packed, kmat):
    return pl.pallas_call(
        _prep_body,
        grid=(GRID,),
        in_specs=[
            pl.BlockSpec((BP, 8), lambda i: (i, 0)),
            pl.BlockSpec((BP, 128), lambda i: (i, 0)),
            pl.BlockSpec((8, 128), lambda i: (0, 0)),
        ],
        out_specs=[
            pl.BlockSpec((BP, 128), lambda i: (i, 0)),
            pl.BlockSpec((BP, 128), lambda i: (i, 0)),
        ],
        out_shape=[
            jax.ShapeDtypeStruct((NPK, 128), jnp.float32),
            jax.ShapeDtypeStruct((NPK, 128), jnp.float32),
        ],
    )(dinv8, x_packed, kmat)


def _mid_body(agg_ref, dinv_ref, m_ref, w2_ref, b1_ref, out_ref):
    a = agg_ref[...]
    s = dinv_ref[...] * (a[0] + a[1])
    hs = []
    for q in range(4):
        z = lax.dot_general(s, m_ref[q], (((1,), (0,)), ((), ())),
                            preferred_element_type=jnp.float32) + b1_ref[...]
        hs.append(jnp.maximum(z, 0.0))
    hcat = jnp.concatenate(hs, axis=1)
    p = lax.dot_general(hcat, w2_ref[...], (((1,), (0,)), ((), ())),
                        preferred_element_type=jnp.float32)
    out_ref[...] = dinv_ref[...] * p


def _mid_call(agg1, dinv16, mstack, w2pp, b1row):
    return pl.pallas_call(
        _mid_body,
        grid=(GRID,),
        in_specs=[
            pl.BlockSpec((NC, BP, 128), lambda i: (0, i, 0)),
            pl.BlockSpec((BP, 128), lambda i: (i, 0)),
            pl.BlockSpec((4, 128, 128), lambda i: (0, 0, 0)),
            pl.BlockSpec((512, 128), lambda i: (0, 0)),
            pl.BlockSpec((1, 128), lambda i: (0, 0)),
        ],
        out_specs=pl.BlockSpec((BP, 128), lambda i: (i, 0)),
        out_shape=jax.ShapeDtypeStruct((NPK, 128), jnp.float32),
    )(agg1, dinv16, mstack, w2pp, b1row)


def _final_body(agg_ref, dinv_ref, k8_ref, sel_ref, b2_ref, out_ref):
    a = agg_ref[...]
    d8 = lax.dot_general(dinv_ref[...], k8_ref[...], (((1,), (0,)), ((), ())),
                         preferred_element_type=jnp.float32)
    y = lax.dot_general(d8 * (a[0] + a[1]), sel_ref[...],
                        (((1,), (0,)), ((), ())),
                        preferred_element_type=jnp.float32)
    out_ref[...] = y + b2_ref[...]


def _final_call(agg2, dinv16c, k8, sel, b2row32):
    npf = N_ACC // 16
    bpf = npf // GRID
    return pl.pallas_call(
        _final_body,
        grid=(GRID,),
        in_specs=[
            pl.BlockSpec((NC, bpf, 128), lambda i: (0, i, 0)),
            pl.BlockSpec((bpf, 16), lambda i: (i, 0)),
            pl.BlockSpec((16, 128), lambda i: (0, 0)),
            pl.BlockSpec((128, 32), lambda i: (0, 0)),
            pl.BlockSpec((1, 32), lambda i: (0, 0)),
        ],
        out_specs=pl.BlockSpec((bpf, 32), lambda i: (i, 0)),
        out_shape=jax.ShapeDtypeStruct((npf, 32), jnp.float32),
    )(agg2, dinv16c, k8, sel, b2row32)


def kernel(x, edge_index, W1, b1, W2, b2):
    loop = jnp.arange(N, dtype=jnp.int32)
    srcA = jnp.full((E_PAD,), N, jnp.int32).at[:E].set(edge_index[0]) \
        .at[E:E + N].set(loop)
    dstA = jnp.full((E_PAD,), N, jnp.int32).at[:E].set(edge_index[1]) \
        .at[E:E + N].set(loop)
    src1_2d = srcA.reshape(-1, 128)
    src2_2d = (srcA * 2).reshape(-1, 128)
    dst2d = dstA.reshape(-1, 128)

    x_packed = jnp.pad(x.reshape(N // 8, 128), ((0, NPK - N // 8), (0, 0)))

    kmat = jnp.kron(jnp.eye(8, dtype=jnp.float32),
                    jnp.ones((1, 16), jnp.float32))
    mstack = jnp.zeros((4, 128, 128), jnp.float32)
    for q in range(4):
        mstack = mstack.at[q, 32 * q:32 * q + 16, 0:64].set(W1)
        mstack = mstack.at[q, 32 * q + 16:32 * q + 32, 64:128].set(W1)
    w2pp = jnp.zeros((512, 128), jnp.float32)
    for q in range(4):
        for c in range(2):
            a_ = 2 * q + c
            w2pp = w2pp.at[128 * q + 64 * c:128 * q + 64 * c + 64,
                           16 * a_:16 * a_ + 2].set(W2)
    k8 = jnp.kron(jnp.eye(16, dtype=jnp.float32),
                  jnp.ones((1, 8), jnp.float32))
    sel = jnp.zeros((128, 32), jnp.float32)
    for a_ in range(16):
        for b_ in range(2):
            sel = sel.at[8 * a_ + b_, 2 * a_ + b_].set(1.0)
    b1row = jnp.tile(b1, 2).reshape(1, 128)
    b2row32 = jnp.tile(b2, 16).reshape(1, 32)

    ones128 = jnp.ones((128,), jnp.float32)
    zeros3200 = jnp.zeros((3200,), jnp.float32)
    zrows16 = jnp.zeros((ZCH, D_IN), jnp.float32)
    zrows8 = jnp.zeros((ZCH, 8), jnp.float32)

    dst_raw = edge_index[1].reshape(RROWS, 128)
    degp = _deg_kernel(dst_raw, ones128, zeros3200)
    dinvf = _dinv_call(degp.reshape(NC, NDG, 128))
    dinv16, xp = _prep_call(dinvf.reshape(NPK, 8), x_packed, kmat)

    agg1 = _agg_x(src1_2d, dst2d, xp.reshape(N_ACC, D_IN), zrows16)
    P2 = _mid_call(agg1.reshape(NC, NPK, 128), dinv16, mstack, w2pp, b1row)

    agg2 = _agg_p(src2_2d, dst2d, P2.reshape(2 * N_ACC, 8), zrows8)
    out = _final_call(agg2.reshape(NC, N_ACC // 16, 128),
                      dinvf.reshape(N_ACC // 16, 16), k8, sel, b2row32)
    return out[:N // 16].reshape(N, 2)
